# baseline (speedup 1.0000x reference)
import functools

import jax
import jax.numpy as jnp
from jax import lax
from jax.experimental import pallas as pl
from jax.experimental.pallas import tpu as pltpu
from jax.experimental.pallas import tpu_sc as plsc

B = 16384
D = 32

_INFO = plsc.get_sparse_core_info()
NC = _INFO.num_cores
NS = _INFO.num_subcores
NW = NC * NS
CHUNK = B // NW
JCH = 128
NJ = CHUNK // JCH


@functools.partial(
    pl.kernel,
    out_type=(
        jax.ShapeDtypeStruct((B, D), jnp.bfloat16),
        jax.ShapeDtypeStruct((B, D), jnp.bfloat16),
        jax.ShapeDtypeStruct((B,), jnp.float32),
        jax.ShapeDtypeStruct((B,), jnp.float32),
    ),
    mesh=plsc.VectorSubcoreMesh(core_axis_name="c", subcore_axis_name="s"),
    compiler_params=pltpu.CompilerParams(
        needs_layout_passes=False, use_tc_tiling_on_sc=False),
    scratch_types=[
        pltpu.VMEM((CHUNK,), jnp.int32),
        pltpu.VMEM((CHUNK,), jnp.int32),
        pltpu.VMEM((CHUNK, D), jnp.bfloat16),
        pltpu.VMEM((CHUNK, D), jnp.bfloat16),
        pltpu.VMEM((CHUNK,), jnp.float32),
        pltpu.VMEM((CHUNK,), jnp.float32),
        pltpu.SemaphoreType.DMA,
        pltpu.SemaphoreType.DMA,
        pltpu.SemaphoreType.DMA,
    ],
)
def _gather_sc(u_hbm, i_hbm, ub_hbm, uv_hbm, ib_hbm, iv_hbm,
               vu_out, vi_out, bu_out, bi_out,
               idx_u, idx_i, vu, vi, bu, bi, sem_u, sem_i, sem_b):
    wid = lax.axis_index("s") * NC + lax.axis_index("c")
    base = pl.multiple_of(wid * CHUNK, CHUNK)

    pltpu.sync_copy(u_hbm.at[pl.ds(base, CHUNK)], idx_u)
    pltpu.sync_copy(i_hbm.at[pl.ds(base, CHUNK)], idx_i)

    copies = []
    for j in range(NJ):
        sl = pl.ds(j * JCH, JCH)
        copies.append(
            pltpu.async_copy(uv_hbm.at[idx_u.at[sl]], vu.at[sl], sem_u))
        copies.append(
            pltpu.async_copy(iv_hbm.at[idx_i.at[sl]], vi.at[sl], sem_i))
        copies.append(
            pltpu.async_copy(ub_hbm.at[idx_u.at[sl]], bu.at[sl], sem_b))
        copies.append(
            pltpu.async_copy(ib_hbm.at[idx_i.at[sl]], bi.at[sl], sem_b))
    for c in copies:
        c.wait()

    pltpu.sync_copy(vu, vu_out.at[pl.ds(base, CHUNK)])
    pltpu.sync_copy(vi, vi_out.at[pl.ds(base, CHUNK)])
    pltpu.sync_copy(bu, bu_out.at[pl.ds(base, CHUNK)])
    pltpu.sync_copy(bi, bi_out.at[pl.ds(base, CHUNK)])


def _dense_tc(c_ref, vu_ref, vi_ref, bu_ref, bi_ref, f_ref, o_ref):
    c0 = c_ref[0]
    c1 = c_ref[1]
    vu = vu_ref[...].astype(jnp.float32)
    vi = vi_ref[...].astype(jnp.float32)
    intx = jnp.sum(vu * vi, axis=1)
    o_ref[...] = (intx + bu_ref[...] + bi_ref[...]
                  + c1 * jnp.log(f_ref[...]) + c0)


def kernel(u, i, f, glob_bias, user_bias, user_vec, item_bias, item_vec,
           poly_W, poly_b):
    u = jnp.squeeze(u).astype(jnp.int32)
    i = jnp.squeeze(i).astype(jnp.int32)
    f = jnp.squeeze(f).astype(jnp.float32)

    uvb = user_vec.astype(jnp.bfloat16)
    ivb = item_vec.astype(jnp.bfloat16)

    vu_g, vi_g, bu_g, bi_g = _gather_sc(
        u, i, user_bias, uvb, item_bias, ivb)

    c = jnp.stack([poly_b[0] + glob_bias[0],
                   poly_W[0, 0] + 2.0 * poly_W[0, 1]])

    return pl.pallas_call(
        _dense_tc,
        out_shape=jax.ShapeDtypeStruct((B,), jnp.float32),
        in_specs=[pl.BlockSpec(memory_space=pltpu.SMEM)]
        + [pl.BlockSpec(memory_space=pltpu.VMEM)] * 5,
        out_specs=pl.BlockSpec(memory_space=pltpu.VMEM),
    )(c, vu_g, vi_g, bu_g, bi_g, f)


# speedup vs baseline: 1.1671x; 1.1671x over previous
import functools

import jax
import jax.numpy as jnp
from jax import lax
from jax.experimental import pallas as pl
from jax.experimental.pallas import tpu as pltpu
from jax.experimental.pallas import tpu_sc as plsc

B = 16384
D = 32

_INFO = plsc.get_sparse_core_info()
NC = _INFO.num_cores
NS = _INFO.num_subcores
NW = NC * NS
CHUNK = B // NW
JCH = 128
NJ = CHUNK // JCH


@functools.partial(
    pl.kernel,
    out_type=(
        jax.ShapeDtypeStruct((B, D), jnp.float32),
        jax.ShapeDtypeStruct((B,), jnp.float32),
    ),
    mesh=plsc.VectorSubcoreMesh(core_axis_name="c", subcore_axis_name="s"),
    compiler_params=pltpu.CompilerParams(
        needs_layout_passes=False, use_tc_tiling_on_sc=False),
    scratch_types=[
        pltpu.VMEM((CHUNK,), jnp.int32),
        pltpu.VMEM((CHUNK, D), jnp.float32),
        pltpu.VMEM((CHUNK,), jnp.float32),
        pltpu.SemaphoreType.DMA,
        pltpu.SemaphoreType.DMA,
    ],
)
def _gather_sc(idx_hbm, bias_hbm, vec_hbm, rows_out, b_out,
               idx_v, rows, bias, sem_r, sem_b):
    wid = lax.axis_index("s") * NC + lax.axis_index("c")
    base = pl.multiple_of(wid * CHUNK, CHUNK)

    pltpu.sync_copy(idx_hbm.at[pl.ds(base, CHUNK)], idx_v)

    copies = []
    for j in range(NJ):
        sl = pl.ds(j * JCH, JCH)
        copies.append(
            pltpu.async_copy(vec_hbm.at[idx_v.at[sl]], rows.at[sl], sem_r))
        copies.append(
            pltpu.async_copy(bias_hbm.at[idx_v.at[sl]], bias.at[sl], sem_b))
    for c in copies:
        c.wait()

    pltpu.sync_copy(rows, rows_out.at[pl.ds(base, CHUNK)])
    pltpu.sync_copy(bias, b_out.at[pl.ds(base, CHUNK)])


def _dense_tc(c_ref, vu_ref, vi_ref, bu_ref, bi_ref, f_ref, o_ref):
    c0 = c_ref[0]
    c1 = c_ref[1]
    intx = jnp.sum(vu_ref[...] * vi_ref[...], axis=1)
    o_ref[...] = (intx + bu_ref[...] + bi_ref[...]
                  + c1 * jnp.log(f_ref[...]) + c0)


def kernel(u, i, f, glob_bias, user_bias, user_vec, item_bias, item_vec,
           poly_W, poly_b):
    u = jnp.squeeze(u).astype(jnp.int32)
    i = jnp.squeeze(i).astype(jnp.int32)
    f = jnp.squeeze(f).astype(jnp.float32)

    vu_g, bu_g = _gather_sc(u, user_bias, user_vec)
    vi_g, bi_g = _gather_sc(i, item_bias, item_vec)

    c = jnp.stack([poly_b[0] + glob_bias[0],
                   poly_W[0, 0] + 2.0 * poly_W[0, 1]])

    return pl.pallas_call(
        _dense_tc,
        out_shape=jax.ShapeDtypeStruct((B,), jnp.float32),
        in_specs=[pl.BlockSpec(memory_space=pltpu.SMEM)]
        + [pl.BlockSpec(memory_space=pltpu.VMEM)] * 5,
        out_specs=pl.BlockSpec(memory_space=pltpu.VMEM),
    )(c, vu_g, vi_g, bu_g, bi_g, f)


# speedup vs baseline: 1.1751x; 1.0069x over previous
import functools

import jax
import jax.numpy as jnp
from jax import lax
from jax.experimental import pallas as pl
from jax.experimental.pallas import tpu as pltpu
from jax.experimental.pallas import tpu_sc as plsc

B = 16384
D = 32

_INFO = plsc.get_sparse_core_info()
NC = _INFO.num_cores
NS = _INFO.num_subcores
NW = NC * NS
CHUNK = B // NW
JCH = 128
NJ = CHUNK // JCH


@functools.partial(
    pl.kernel,
    out_type=(
        jax.ShapeDtypeStruct((B, D), jnp.float32),
        jax.ShapeDtypeStruct((B,), jnp.float32),
    ),
    mesh=plsc.VectorSubcoreMesh(core_axis_name="c", subcore_axis_name="s"),
    compiler_params=pltpu.CompilerParams(
        needs_layout_passes=False, use_tc_tiling_on_sc=False),
    scratch_types=[
        pltpu.VMEM((CHUNK,), jnp.int32),
        pltpu.VMEM((CHUNK, D), jnp.float32),
        pltpu.VMEM((CHUNK,), jnp.float32),
        pltpu.SemaphoreType.DMA,
        pltpu.SemaphoreType.DMA,
    ],
)
def _gather_sc(idx_hbm, bias_hbm, vec_hbm, rows_out, b_out,
               idx_v, rows, bias, sem_r, sem_b):
    wid = lax.axis_index("s") * NC + lax.axis_index("c")
    base = pl.multiple_of(wid * CHUNK, CHUNK)

    pltpu.sync_copy(idx_hbm.at[pl.ds(base, CHUNK)], idx_v)

    copies = []
    for j in range(NJ):
        sl = pl.ds(j * JCH, JCH)
        copies.append(
            pltpu.async_copy(vec_hbm.at[idx_v.at[sl]], rows.at[sl], sem_r))
        copies.append(
            pltpu.async_copy(bias_hbm.at[idx_v.at[sl]], bias.at[sl], sem_b))
    for c in copies:
        c.wait()

    pltpu.sync_copy(rows, rows_out.at[pl.ds(base, CHUNK)])
    pltpu.sync_copy(bias, b_out.at[pl.ds(base, CHUNK)])


def _dense_tc(c_ref, vu_ref, vi_ref, bu_ref, bi_ref, f_ref, o_ref):
    c0 = c_ref[0]
    c1 = c_ref[1]
    intx = jnp.sum(vu_ref[...] * vi_ref[...], axis=1)
    o_ref[...] = (intx + bu_ref[...] + bi_ref[...]
                  + c1 * jnp.log(f_ref[...]) + c0)


def kernel(u, i, f, glob_bias, user_bias, user_vec, item_bias, item_vec,
           poly_W, poly_b):
    u = jnp.squeeze(u).astype(jnp.int32)
    i = jnp.squeeze(i).astype(jnp.int32)
    f = jnp.squeeze(f).astype(jnp.float32)

    vu_g, bu_g = _gather_sc(u, user_bias, user_vec)
    vi_g, bi_g = _gather_sc(i, item_bias, item_vec)

    c = jnp.stack([poly_b[0] + glob_bias[0],
                   poly_W[0, 0] + 2.0 * poly_W[0, 1]])

    blk = 2048
    vspec = pl.BlockSpec((blk, D), lambda g: (g, 0))
    sspec = pl.BlockSpec((blk,), lambda g: (g,))
    return pl.pallas_call(
        _dense_tc,
        grid=(B // blk,),
        out_shape=jax.ShapeDtypeStruct((B,), jnp.float32),
        in_specs=[pl.BlockSpec(memory_space=pltpu.SMEM),
                  vspec, vspec, sspec, sspec, sspec],
        out_specs=sspec,
    )(c, vu_g, vi_g, bu_g, bi_g, f)
